# baseline (speedup 1.0000x reference)
import jax
import jax.numpy as jnp
from jax import lax
from jax.experimental import pallas as pl
from jax.experimental.pallas import tpu as pltpu
from jax.experimental.pallas import tpu_sc as plsc

NC = 2
NS = 16
NW = NC * NS

D = 32
IB = 128
L = 16


def _sc_gather(xt, table, n, s):
    nblk = n // IB
    units = s * nblk
    upw = units // NW
    mesh = plsc.VectorSubcoreMesh(core_axis_name="c", subcore_axis_name="s")

    @pl.kernel(
        out_type=jax.ShapeDtypeStruct((s, D // 8, nblk, 8, IB), jnp.float32),
        mesh=mesh,
        compiler_params=pltpu.CompilerParams(
            use_tc_tiling_on_sc=False, needs_layout_passes=False
        ),
        scratch_types=[
            pltpu.VMEM((2, IB), jnp.int32),
            pltpu.VMEM((2, IB, D + 1), jnp.float32),
            pltpu.VMEM((D // 8, 8, IB), jnp.float32),
            pltpu.SemaphoreType.DMA((2,)),
            pltpu.SemaphoreType.DMA,
        ],
    )
    def k(x_hbm, table_hbm, out_hbm, gidx_v, rows_v, blk_v, gsem, osem):
        cid = lax.axis_index("c")
        sid = lax.axis_index("s")
        wid = sid * NC + cid
        u0 = wid * upw

        def stage_and_fire(u, slot):
            j = u // nblk
            b = u - j * nblk
            pltpu.sync_copy(x_hbm.at[j, pl.ds(b * IB, IB)], gidx_v.at[slot])
            pltpu.async_copy(table_hbm.at[gidx_v.at[slot]], rows_v.at[slot],
                             gsem.at[slot])

        def consume(u, slot, wait_out):
            j = u // nblk
            b = u - j * nblk
            pltpu.make_async_copy(
                table_hbm.at[gidx_v.at[slot]], rows_v.at[slot], gsem.at[slot]
            ).wait()
            if wait_out:
                pltpu.make_async_copy(
                    blk_v, out_hbm.at[0, :, 0], osem
                ).wait()
            for kk in range(IB // L):
                row16 = lax.iota(jnp.int32, L) + kk * L
                for d in range(D):
                    col16 = jnp.full((L,), d, jnp.int32)
                    blk_v[d // 8, d % 8, pl.ds(kk * L, L)] = plsc.load_gather(
                        rows_v.at[slot], [row16, col16]
                    )
            pltpu.async_copy(blk_v, out_hbm.at[j, :, b], osem)

        stage_and_fire(u0, 0)
        stage_and_fire(u0 + 1, 1)
        consume(u0, 0, wait_out=False)

        def pair(p, carry):
            u = u0 + 2 * p
            stage_and_fire(u + 2, 0)
            consume(u + 1, 1, wait_out=True)
            stage_and_fire(u + 3, 1)
            consume(u + 2, 0, wait_out=True)
            return carry

        lax.fori_loop(0, (upw - 2) // 2, pair, 0)

        consume(u0 + upw - 1, 1, wait_out=True)
        pltpu.make_async_copy(
            blk_v, out_hbm.at[0, :, 0], osem
        ).wait()

    return k(xt, table)


def kernel(x, table):
    n, s = x.shape
    xt = x.astype(jnp.int32).T
    tablep = jnp.pad(table, ((0, 0), (0, 1)))
    out = _sc_gather(xt, tablep, n, s)
    return out.transpose(2, 4, 0, 1, 3).reshape(n, s, D)


# speedup vs baseline: 1.4201x; 1.4201x over previous
import jax
import jax.numpy as jnp
from jax import lax
from jax.experimental import pallas as pl
from jax.experimental.pallas import tpu as pltpu
from jax.experimental.pallas import tpu_sc as plsc

NC = 2
NS = 16
NW = NC * NS

D = 32
IB = 128
L = 16


def _sc_gather(xt, table, n, s):
    nblk = n // IB
    units = s * nblk
    upw = units // NW
    mesh = plsc.VectorSubcoreMesh(core_axis_name="c", subcore_axis_name="s")

    @pl.kernel(
        out_type=jax.ShapeDtypeStruct((s, D // 8, nblk, 8, IB), jnp.float32),
        mesh=mesh,
        compiler_params=pltpu.CompilerParams(
            use_tc_tiling_on_sc=False, needs_layout_passes=False
        ),
        scratch_types=[
            pltpu.VMEM((2, IB), jnp.int32),
            pltpu.VMEM((2, IB, D), jnp.float32),
            pltpu.VMEM((D // 8, 8, IB), jnp.float32),
            pltpu.SemaphoreType.DMA((2,)),
            pltpu.SemaphoreType.DMA,
        ],
    )
    def k(x_hbm, table_hbm, out_hbm, gidx_v, rows_v, blk_v, gsem, osem):
        cid = lax.axis_index("c")
        sid = lax.axis_index("s")
        wid = sid * NC + cid
        u0 = wid * upw

        def stage_and_fire(u, slot):
            j = u // nblk
            b = u - j * nblk
            pltpu.sync_copy(x_hbm.at[j, pl.ds(b * IB, IB)], gidx_v.at[slot])
            pltpu.async_copy(table_hbm.at[gidx_v.at[slot]], rows_v.at[slot],
                             gsem.at[slot])

        def consume(u, slot, wait_out):
            j = u // nblk
            b = u - j * nblk
            pltpu.make_async_copy(
                table_hbm.at[gidx_v.at[slot]], rows_v.at[slot], gsem.at[slot]
            ).wait()
            if wait_out:
                pltpu.make_async_copy(
                    blk_v, out_hbm.at[0, :, 0], osem
                ).wait()
            for kk in range(IB // L):
                i16 = lax.iota(jnp.int32, L) + kk * L
                for d in range(D):
                    c16 = (i16 + d) & (D - 1)
                    v = plsc.load_gather(rows_v.at[slot], [i16, c16])
                    plsc.store_scatter(blk_v, [c16 >> 3, c16 & 7, i16], v)
            pltpu.async_copy(blk_v, out_hbm.at[j, :, b], osem)

        stage_and_fire(u0, 0)
        stage_and_fire(u0 + 1, 1)
        consume(u0, 0, wait_out=False)

        def pair(p, carry):
            u = u0 + 2 * p
            stage_and_fire(u + 2, 0)
            consume(u + 1, 1, wait_out=True)
            stage_and_fire(u + 3, 1)
            consume(u + 2, 0, wait_out=True)
            return carry

        lax.fori_loop(0, (upw - 2) // 2, pair, 0)

        consume(u0 + upw - 1, 1, wait_out=True)
        pltpu.make_async_copy(
            blk_v, out_hbm.at[0, :, 0], osem
        ).wait()

    return k(xt, table)


def kernel(x, table):
    n, s = x.shape
    xt = x.astype(jnp.int32).T
    out = _sc_gather(xt, table, n, s)
    return out.transpose(2, 4, 0, 1, 3).reshape(n, s, D)
